# baseline (speedup 1.0000x reference)
import jax
import jax.numpy as jnp
from jax import lax
from jax.experimental import pallas as pl
from jax.experimental.pallas import tpu as pltpu
from jax.experimental.pallas import tpu_sc as plsc

B = 16384
D = 64
K = 4
NC = 2
NS = 16
NW = NC * NS
B_PER_W = B // NW
CHUNK = 128
N_CHUNKS = B_PER_W // CHUNK

_PAIR_BASE = (0, 500000, 550000, 600000)

_DNUMS = lax.GatherDimensionNumbers(
    offset_dims=(), collapsed_slice_dims=(0,), start_index_map=(0,))


def _eges_body(sku_hbm, brand_hbm, shop_hbm, cate_hbm,
               tab_hbm, at_hbm, out_hbm,
               idx_v, pidx_v, aidx_v, rows_v, logit_v, out_v, sem):
    wid = lax.axis_index("s") * NC + lax.axis_index("c")
    base = pl.multiple_of(wid * B_PER_W, B_PER_W)

    feats = (sku_hbm, brand_hbm, shop_hbm, cate_hbm)
    for f in range(K):
        pltpu.sync_copy(feats[f].at[pl.ds(base, B_PER_W)], idx_v.at[f])

    def prep(i, carry):
        sl = pl.ds(i * 16, 16)
        for f in range(K):
            pidx_v[f, sl] = (lax.shift_right_logical(idx_v[f, sl], 1)
                             + _PAIR_BASE[f])
        sv = idx_v[0, sl] * K
        for k in range(K):
            aidx_v[k, sl] = sv + k
        return carry

    lax.fori_loop(0, B_PER_W // 16, prep, 0)

    for c in range(N_CHUNKS):
        off = c * CHUNK
        copies = []
        for f in range(K):
            copies.append(pltpu.async_copy(
                tab_hbm.at[pidx_v.at[f, pl.ds(off, CHUNK)]],
                rows_v.at[f], sem))
        for k in range(K):
            copies.append(pltpu.async_copy(
                at_hbm.at[aidx_v.at[k, pl.ds(off, CHUNK)]],
                logit_v.at[k], sem))
        for cp in copies:
            cp.wait()

        def group(g, carry):
            tbase = g * 16
            e = [jnp.exp(logit_v[k, pl.ds(tbase, 16)]) for k in range(K)]
            s = (e[0] + e[1]) + (e[2] + e[3])
            w16 = [e[k] / s for k in range(K)]
            hoff = [(idx_v[f, pl.ds(off + tbase, 16)] & 1) * 64
                    for f in range(K)]
            for l in range(16):
                t = tbase + l
                acc = [None] * (D // 16)
                for k in range(K):
                    wk = _bcast_lane(w16[k], l)
                    hk = hoff[k][l]
                    for j in range(D // 16):
                        term = rows_v[k, t, pl.ds(hk + j * 16, 16)] * wk
                        acc[j] = term if k == 0 else acc[j] + term
                orow = 8 * g + (l >> 1)
                ocol = (l & 1) * 64
                for j in range(D // 16):
                    out_v[orow, pl.ds(ocol + j * 16, 16)] = acc[j]
            return carry

        lax.fori_loop(0, CHUNK // 16, group, 0)

        obase = pl.multiple_of((base + off) // 2, CHUNK // 2)
        pltpu.sync_copy(out_v, out_hbm.at[pl.ds(obase, CHUNK // 2)])


def _bcast_lane(v, lane):
    idx = jnp.full((16,), lane, jnp.int32)
    return lax.gather(v, idx[:, None], _DNUMS, (1,),
                      mode=lax.GatherScatterMode.PROMISE_IN_BOUNDS)


@jax.jit
def _eges(sku, brand, shop, cate, tab2, attn_flat):
    out2 = pl.kernel(
        _eges_body,
        mesh=plsc.VectorSubcoreMesh(core_axis_name="c", subcore_axis_name="s"),
        compiler_params=pltpu.CompilerParams(use_tc_tiling_on_sc=False),
        out_type=jax.ShapeDtypeStruct((B // 2, 2 * D), jnp.float32),
        scratch_types=[
            pltpu.VMEM((K, B_PER_W), jnp.int32),
            pltpu.VMEM((K, B_PER_W), jnp.int32),
            pltpu.VMEM((K, B_PER_W), jnp.int32),
            pltpu.VMEM((K, CHUNK, 2 * D), jnp.float32),
            pltpu.VMEM((K, CHUNK), jnp.float32),
            pltpu.VMEM((CHUNK // 2, 2 * D), jnp.float32),
            pltpu.SemaphoreType.DMA,
        ],
    )(sku, brand, shop, cate, tab2, attn_flat)
    return out2.reshape(B, D)


def kernel(sku_id, brand, shop, cate, emb_sku, emb_brand, emb_shop, emb_cate,
           attn_tab):
    tab2 = jnp.concatenate(
        [emb_sku.reshape(-1, 2 * D), emb_brand.reshape(-1, 2 * D),
         emb_shop.reshape(-1, 2 * D), emb_cate.reshape(-1, 2 * D)], axis=0)
    return _eges(sku_id.astype(jnp.int32), brand.astype(jnp.int32),
                 shop.astype(jnp.int32), cate.astype(jnp.int32),
                 tab2, attn_tab.reshape(-1))


# speedup vs baseline: 1.1501x; 1.1501x over previous
import jax
import jax.numpy as jnp
from jax import lax
from jax.experimental import pallas as pl
from jax.experimental.pallas import tpu as pltpu
from jax.experimental.pallas import tpu_sc as plsc

B = 16384
D = 64
K = 4
NC = 2
NS = 16
NW = NC * NS
B_PER_W = B // NW
CHUNK = 128
N_CHUNKS = B_PER_W // CHUNK

_DNUMS = lax.GatherDimensionNumbers(
    offset_dims=(), collapsed_slice_dims=(0,), start_index_map=(0,))


def _eges_body(sku_hbm, brand_hbm, shop_hbm, cate_hbm,
               es_hbm, eb_hbm, eh_hbm, ec_hbm, at_hbm, out_hbm,
               idx_v, pidx_v, aidx_v, rows_v, logit_v, out_v, sem):
    wid = lax.axis_index("s") * NC + lax.axis_index("c")
    base = pl.multiple_of(wid * B_PER_W, B_PER_W)

    feats = (sku_hbm, brand_hbm, shop_hbm, cate_hbm)
    for f in range(K):
        pltpu.sync_copy(feats[f].at[pl.ds(base, B_PER_W)], idx_v.at[f])

    def prep(i, carry):
        sl = pl.ds(i * 16, 16)
        for f in range(K):
            pidx_v[f, sl] = lax.shift_right_logical(idx_v[f, sl], 1)
        sv = idx_v[0, sl] * K
        for k in range(K):
            aidx_v[k, sl] = sv + k
        return carry

    lax.fori_loop(0, B_PER_W // 16, prep, 0)

    tables = (es_hbm, eb_hbm, eh_hbm, ec_hbm)

    for c in range(N_CHUNKS):
        off = c * CHUNK
        copies = []
        for f in range(K):
            copies.append(pltpu.async_copy(
                tables[f].at[pidx_v.at[f, pl.ds(off, CHUNK)]],
                rows_v.at[f], sem))
        for k in range(K):
            copies.append(pltpu.async_copy(
                at_hbm.at[aidx_v.at[k, pl.ds(off, CHUNK)]],
                logit_v.at[k], sem))
        for cp in copies:
            cp.wait()

        def group(g, carry):
            tbase = g * 16
            e = [jnp.exp(logit_v[k, pl.ds(tbase, 16)]) for k in range(K)]
            s = (e[0] + e[1]) + (e[2] + e[3])
            w16 = [e[k] / s for k in range(K)]
            hoff = [(idx_v[f, pl.ds(off + tbase, 16)] & 1) * 64
                    for f in range(K)]
            for l in range(16):
                t = tbase + l
                acc = [None] * (D // 16)
                for k in range(K):
                    wk = _bcast_lane(w16[k], l)
                    hk = hoff[k][l]
                    for j in range(D // 16):
                        term = rows_v[k, t, pl.ds(hk + j * 16, 16)] * wk
                        acc[j] = term if k == 0 else acc[j] + term
                orow = 8 * g + (l >> 1)
                ocol = (l & 1) * 64
                for j in range(D // 16):
                    out_v[orow, pl.ds(ocol + j * 16, 16)] = acc[j]
            return carry

        lax.fori_loop(0, CHUNK // 16, group, 0)

        obase = pl.multiple_of((base + off) // 2, CHUNK // 2)
        pltpu.sync_copy(out_v, out_hbm.at[pl.ds(obase, CHUNK // 2)])


def _bcast_lane(v, lane):
    idx = jnp.full((16,), lane, jnp.int32)
    return lax.gather(v, idx[:, None], _DNUMS, (1,),
                      mode=lax.GatherScatterMode.PROMISE_IN_BOUNDS)


@jax.jit
def _eges(sku, brand, shop, cate, es2, eb2, eh2, ec2, attn_flat):
    out2 = pl.kernel(
        _eges_body,
        mesh=plsc.VectorSubcoreMesh(core_axis_name="c", subcore_axis_name="s"),
        compiler_params=pltpu.CompilerParams(use_tc_tiling_on_sc=False),
        out_type=jax.ShapeDtypeStruct((B // 2, 2 * D), jnp.float32),
        scratch_types=[
            pltpu.VMEM((K, B_PER_W), jnp.int32),
            pltpu.VMEM((K, B_PER_W), jnp.int32),
            pltpu.VMEM((K, B_PER_W), jnp.int32),
            pltpu.VMEM((K, CHUNK, 2 * D), jnp.float32),
            pltpu.VMEM((K, CHUNK), jnp.float32),
            pltpu.VMEM((CHUNK // 2, 2 * D), jnp.float32),
            pltpu.SemaphoreType.DMA,
        ],
    )(sku, brand, shop, cate, es2, eb2, eh2, ec2, attn_flat)
    return out2.reshape(B, D)


def kernel(sku_id, brand, shop, cate, emb_sku, emb_brand, emb_shop, emb_cate,
           attn_tab):
    return _eges(sku_id.astype(jnp.int32), brand.astype(jnp.int32),
                 shop.astype(jnp.int32), cate.astype(jnp.int32),
                 emb_sku.reshape(-1, 2 * D),
                 emb_brand.reshape(-1, 2 * D),
                 emb_shop.reshape(-1, 2 * D),
                 emb_cate.reshape(-1, 2 * D),
                 attn_tab.reshape(-1))
